# Initial kernel scaffold; baseline (speedup 1.0000x reference)
import jax
import jax.numpy as jnp
from jax.experimental import pallas as pl


def kernel(x, edge_index, batch, W1, b1, W2, b2, Wf1, bf1, Wf2, bf2):
    raise NotImplementedError("write your pallas kernel here")


# speedup vs baseline: 29.5033x; 29.5033x over previous
import functools

import jax
import jax.numpy as jnp
from jax import lax
from jax.experimental import pallas as pl
from jax.experimental.pallas import tpu as pltpu
from jax.experimental.pallas import tpu_sc as plsc

NC = 2
NS = 16
NW = NC * NS
B = 125


def _sc_mesh():
    return plsc.VectorSubcoreMesh(core_axis_name="c", subcore_axis_name="s",
                                  num_cores=NC, num_subcores=NS)


DEGW = 16


def _make_deg_kernel(n_chunks, chunks_per_tile, np_pad):
    rows = np_pad // NS

    @functools.partial(
        pl.kernel,
        mesh=_sc_mesh(),
        compiler_params=pltpu.CompilerParams(use_tc_tiling_on_sc=False),
        out_type=jax.ShapeDtypeStruct((NC, np_pad, DEGW), jnp.float32),
        scratch_types=[
            pltpu.VMEM((chunks_per_tile, B), jnp.int32),
            pltpu.VMEM((B, DEGW), jnp.float32),
            pltpu.VMEM_SHARED((np_pad, DEGW), jnp.float32),
        ],
    )
    def deg_kernel(dst_hbm, ones_hbm, zeros_hbm, out_hbm, idx_v, ones_v, acc_sh):
        c = lax.axis_index("c")
        s = lax.axis_index("s")
        wid = s * NC + c
        pltpu.sync_copy(zeros_hbm.at[pl.ds(s * rows, rows)],
                        acc_sh.at[pl.ds(s * rows, rows)])
        pltpu.sync_copy(ones_hbm, ones_v)
        pltpu.sync_copy(dst_hbm.at[pl.ds(wid * chunks_per_tile, chunks_per_tile)],
                        idx_v)
        plsc.subcore_barrier()

        def body(i, carry):
            pltpu.sync_copy(ones_v, acc_sh.at[idx_v.at[i]], add=True)
            return carry

        lax.fori_loop(0, chunks_per_tile, body, 0)
        plsc.subcore_barrier()
        pltpu.sync_copy(acc_sh.at[pl.ds(s * rows, rows)],
                        out_hbm.at[c, pl.ds(s * rows, rows)])

    return deg_kernel


def _make_conv_kernel(feat, n_chunks, chunks_per_tile, np_pad):
    rows = np_pad // NS

    @functools.partial(
        pl.kernel,
        mesh=_sc_mesh(),
        compiler_params=pltpu.CompilerParams(use_tc_tiling_on_sc=False),
        out_type=jax.ShapeDtypeStruct((NC, np_pad, feat), jnp.float32),
        scratch_types=[
            pltpu.VMEM((chunks_per_tile, B), jnp.int32),
            pltpu.VMEM((chunks_per_tile, B), jnp.int32),
            pltpu.VMEM((B, feat), jnp.float32),
            pltpu.VMEM_SHARED((np_pad, feat), jnp.float32),
        ],
    )
    def conv_kernel(table_hbm, src_hbm, dst_hbm, zeros_hbm, out_hbm,
                    idxs_v, idxd_v, rows_v, acc_sh):
        c = lax.axis_index("c")
        s = lax.axis_index("s")
        wid = s * NC + c
        pltpu.sync_copy(zeros_hbm.at[pl.ds(s * rows, rows)],
                        acc_sh.at[pl.ds(s * rows, rows)])
        pltpu.sync_copy(src_hbm.at[pl.ds(wid * chunks_per_tile, chunks_per_tile)],
                        idxs_v)
        pltpu.sync_copy(dst_hbm.at[pl.ds(wid * chunks_per_tile, chunks_per_tile)],
                        idxd_v)
        plsc.subcore_barrier()

        def body(i, carry):
            pltpu.sync_copy(table_hbm.at[idxs_v.at[i]], rows_v)
            pltpu.sync_copy(rows_v, acc_sh.at[idxd_v.at[i]], add=True)
            return carry

        lax.fori_loop(0, chunks_per_tile, body, 0)
        plsc.subcore_barrier()
        pltpu.sync_copy(acc_sh.at[pl.ds(s * rows, rows)],
                        out_hbm.at[c, pl.ds(s * rows, rows)])

    return conv_kernel


def _scale1_body(n, degp_ref, x_ref, w1_ref, xs_ref, dinv_ref):
    dp = degp_ref[...]
    deg = dp[0, :n, :1] + dp[1, :n, :1] + 1.0
    dinv = lax.rsqrt(deg)
    xw = jnp.dot(x_ref[...], w1_ref[...], preferred_element_type=jnp.float32,
                 precision=lax.Precision.HIGHEST)
    xs_ref[...] = xw * dinv
    dinv_ref[...] = dinv


def _mid_body(n, aggp_ref, xs_ref, dinv_ref, b1_ref, w2_ref, x1_ref, xs2_ref):
    a = aggp_ref[...]
    dinv = dinv_ref[...]
    x1 = jnp.maximum((a[0, :n] + a[1, :n] + xs_ref[...]) * dinv + b1_ref[...], 0.0)
    x1_ref[...] = x1
    xs2_ref[...] = jnp.dot(x1, w2_ref[...],
                           preferred_element_type=jnp.float32,
                 precision=lax.Precision.HIGHEST) * dinv


def _final_body(n, g, aggp_ref, xs2_ref, dinv_ref, b2_ref, x1_ref, batch_ref,
                wf1a_ref, wf1b_ref, bf1_ref, wf2_ref, bf2_ref, out_ref):
    a = aggp_ref[...]
    x2 = jnp.maximum(
        (a[0, :n] + a[1, :n] + xs2_ref[...]) * dinv_ref[...] + b2_ref[...], 0.0)
    gid = lax.broadcasted_iota(jnp.int32, (g, 1), 0)
    oh = (gid == batch_ref[...]).astype(jnp.float32)
    sums1 = jnp.dot(oh, x1_ref[...], preferred_element_type=jnp.float32,
                 precision=lax.Precision.HIGHEST)
    sums2 = jnp.dot(oh, x2, preferred_element_type=jnp.float32,
                 precision=lax.Precision.HIGHEST)
    inv_cnt = 1.0 / jnp.maximum(jnp.sum(oh, axis=1, keepdims=True), 1.0)
    pool1 = sums1 * inv_cnt
    pool2 = sums2 * inv_cnt
    h = jnp.maximum(
        jnp.dot(pool1, wf1a_ref[...], preferred_element_type=jnp.float32,
                 precision=lax.Precision.HIGHEST)
        + jnp.dot(pool2, wf1b_ref[...], preferred_element_type=jnp.float32,
                 precision=lax.Precision.HIGHEST)
        + bf1_ref[...], 0.0)
    out_ref[...] = jnp.dot(h, wf2_ref[...],
                           preferred_element_type=jnp.float32,
                 precision=lax.Precision.HIGHEST) + bf2_ref[...]


def kernel(x, edge_index, batch, W1, b1, W2, b2, Wf1, bf1, Wf2, bf2):
    n, d = x.shape
    e = edge_index.shape[1]
    f1 = W1.shape[1]
    f2 = W2.shape[1]
    g = 64
    np_pad = ((n + NS * 8 - 1) // (NS * 8)) * (NS * 8)
    n_chunks = e // B
    chunks_per_tile = n_chunks // NW

    src2d = edge_index[0].reshape(n_chunks, B)
    dst2d = edge_index[1].reshape(n_chunks, B)
    batch2d = batch.reshape(1, n)
    ones_b = jnp.ones((B, DEGW), jnp.float32)
    zeros1 = jnp.zeros((np_pad, DEGW), jnp.float32)
    zeros_f1 = jnp.zeros((np_pad, f1), jnp.float32)
    zeros_f2 = jnp.zeros((np_pad, f2), jnp.float32)

    deg_k = _make_deg_kernel(n_chunks, chunks_per_tile, np_pad)
    degp = deg_k(dst2d, ones_b, zeros1)

    scale1 = pl.pallas_call(
        functools.partial(_scale1_body, n),
        out_shape=(jax.ShapeDtypeStruct((n, f1), jnp.float32),
                   jax.ShapeDtypeStruct((n, 1), jnp.float32)),
    )
    xs1, dinv = scale1(degp, x, W1)

    conv1_k = _make_conv_kernel(f1, n_chunks, chunks_per_tile, np_pad)
    agg1p = conv1_k(xs1, src2d, dst2d, zeros_f1)

    mid = pl.pallas_call(
        functools.partial(_mid_body, n),
        out_shape=(jax.ShapeDtypeStruct((n, f1), jnp.float32),
                   jax.ShapeDtypeStruct((n, f2), jnp.float32)),
    )
    x1, xs2 = mid(agg1p, xs1, dinv, b1.reshape(1, f1), W2)

    conv2_k = _make_conv_kernel(f2, n_chunks, chunks_per_tile, np_pad)
    agg2p = conv2_k(xs2, src2d, dst2d, zeros_f2)

    final = pl.pallas_call(
        functools.partial(_final_body, n, g),
        out_shape=jax.ShapeDtypeStruct((g, 1), jnp.float32),
    )
    out = final(agg2p, xs2, dinv, b2.reshape(1, f2), x1, batch2d,
                Wf1[:f1], Wf1[f1:], bf1.reshape(1, -1), Wf2,
                bf2.reshape(1, 1))
    return out.reshape(-1)


# speedup vs baseline: 43.3574x; 1.4696x over previous
import functools

import jax
import jax.numpy as jnp
from jax import lax
from jax.experimental import pallas as pl
from jax.experimental.pallas import tpu as pltpu
from jax.experimental.pallas import tpu_sc as plsc

NC = 2
NS = 16
NW = NC * NS
B = 125


def _sc_mesh():
    return plsc.VectorSubcoreMesh(core_axis_name="c", subcore_axis_name="s",
                                  num_cores=NC, num_subcores=NS)


DEGW = 16


def _make_deg_kernel(n_chunks, chunks_per_tile, np_pad):
    rows = np_pad // NS

    @functools.partial(
        pl.kernel,
        mesh=_sc_mesh(),
        compiler_params=pltpu.CompilerParams(use_tc_tiling_on_sc=False),
        out_type=jax.ShapeDtypeStruct((NC, np_pad, DEGW), jnp.float32),
        scratch_types=[
            pltpu.VMEM((chunks_per_tile, B), jnp.int32),
            pltpu.VMEM((B, DEGW), jnp.float32),
            pltpu.VMEM_SHARED((np_pad, DEGW), jnp.float32),
            pltpu.SemaphoreType.DMA,
        ],
    )
    def deg_kernel(dst_hbm, ones_hbm, zeros_hbm, out_hbm, idx_v, ones_v, acc_sh,
                   sem):
        c = lax.axis_index("c")
        s = lax.axis_index("s")
        wid = s * NC + c
        pltpu.sync_copy(zeros_hbm.at[pl.ds(s * rows, rows)],
                        acc_sh.at[pl.ds(s * rows, rows)])
        pltpu.sync_copy(ones_hbm, ones_v)
        pltpu.sync_copy(dst_hbm.at[pl.ds(wid * chunks_per_tile, chunks_per_tile)],
                        idx_v)
        plsc.subcore_barrier()

        def fire(i, carry):
            pltpu.async_copy(ones_v, acc_sh.at[idx_v.at[i]], sem, add=True)
            return carry

        lax.fori_loop(0, chunks_per_tile, fire, 0)

        def drain(i, carry):
            pltpu.make_async_copy(ones_v, acc_sh.at[idx_v.at[i]], sem).wait()
            return carry

        lax.fori_loop(0, chunks_per_tile, drain, 0)
        plsc.subcore_barrier()
        pltpu.sync_copy(acc_sh.at[pl.ds(s * rows, rows)],
                        out_hbm.at[c, pl.ds(s * rows, rows)])

    return deg_kernel


NBUF = 4


def _make_conv_kernel(feat, n_chunks, chunks_per_tile, np_pad):
    rows = np_pad // NS
    n_batches = chunks_per_tile // NBUF

    @functools.partial(
        pl.kernel,
        mesh=_sc_mesh(),
        compiler_params=pltpu.CompilerParams(use_tc_tiling_on_sc=False),
        out_type=jax.ShapeDtypeStruct((NC, np_pad, feat), jnp.float32),
        scratch_types=[
            pltpu.VMEM((chunks_per_tile, B), jnp.int32),
            pltpu.VMEM((chunks_per_tile, B), jnp.int32),
            pltpu.VMEM((2, NBUF, B, feat), jnp.float32),
            pltpu.VMEM_SHARED((np_pad, feat), jnp.float32),
            pltpu.SemaphoreType.DMA,
            pltpu.SemaphoreType.DMA,
            pltpu.SemaphoreType.DMA,
        ],
    )
    def conv_kernel(table_hbm, src_hbm, dst_hbm, zeros_hbm, out_hbm,
                    idxs_v, idxd_v, bufs_v, acc_sh, gsem, ssem_a, ssem_b):
        ssems = (ssem_a, ssem_b)
        c = lax.axis_index("c")
        s = lax.axis_index("s")
        wid = s * NC + c
        pltpu.sync_copy(zeros_hbm.at[pl.ds(s * rows, rows)],
                        acc_sh.at[pl.ds(s * rows, rows)])
        pltpu.sync_copy(src_hbm.at[pl.ds(wid * chunks_per_tile, chunks_per_tile)],
                        idxs_v)
        pltpu.sync_copy(dst_hbm.at[pl.ds(wid * chunks_per_tile, chunks_per_tile)],
                        idxd_v)
        plsc.subcore_barrier()

        def g_start(t, grp, b):
            pltpu.async_copy(table_hbm.at[idxs_v.at[t * NBUF + b]],
                             bufs_v.at[grp, b], gsem)

        def g_wait(t, grp, b):
            pltpu.make_async_copy(table_hbm.at[idxs_v.at[t * NBUF + b]],
                                  bufs_v.at[grp, b], gsem).wait()

        def s_start(t, grp, b):
            pltpu.async_copy(bufs_v.at[grp, b],
                             acc_sh.at[idxd_v.at[t * NBUF + b]], ssems[grp],
                             add=True)

        def s_wait(t, grp, b):
            pltpu.make_async_copy(bufs_v.at[grp, b],
                                  acc_sh.at[idxd_v.at[t * NBUF + b]],
                                  ssems[grp]).wait()

        for b in range(NBUF):
            g_start(0, 0, b)

        def outer(t2, carry):
            for p in (0, 1):
                t = t2 * 2 + p
                for b in range(NBUF):
                    g_wait(t, p, b)
                for b in range(NBUF):
                    s_start(t, p, b)

                @pl.when(t >= 1)
                def _():
                    for b in range(NBUF):
                        s_wait(t - 1, 1 - p, b)

                @pl.when(t + 1 < n_batches)
                def _():
                    for b in range(NBUF):
                        g_start(t + 1, 1 - p, b)
            return carry

        lax.fori_loop(0, n_batches // 2, outer, 0)
        for b in range(NBUF):
            s_wait(n_batches - 1, 1, b)
        plsc.subcore_barrier()
        pltpu.sync_copy(acc_sh.at[pl.ds(s * rows, rows)],
                        out_hbm.at[c, pl.ds(s * rows, rows)])

    return conv_kernel


def _scale1_body(n, degp_ref, x_ref, w1_ref, xs_ref, dinv_ref):
    dp = degp_ref[...]
    deg = dp[0, :n, :1] + dp[1, :n, :1] + 1.0
    dinv = lax.rsqrt(deg)
    xw = jnp.dot(x_ref[...], w1_ref[...], preferred_element_type=jnp.float32,
                 precision=lax.Precision.HIGHEST)
    xs_ref[...] = xw * dinv
    dinv_ref[...] = dinv


def _mid_body(n, aggp_ref, xs_ref, dinv_ref, b1_ref, w2_ref, x1_ref, xs2_ref):
    a = aggp_ref[...]
    dinv = dinv_ref[...]
    x1 = jnp.maximum((a[0, :n] + a[1, :n] + xs_ref[...]) * dinv + b1_ref[...], 0.0)
    x1_ref[...] = x1
    xs2_ref[...] = jnp.dot(x1, w2_ref[...],
                           preferred_element_type=jnp.float32,
                 precision=lax.Precision.HIGHEST) * dinv


def _final_body(n, g, aggp_ref, xs2_ref, dinv_ref, b2_ref, x1_ref, batch_ref,
                wf1a_ref, wf1b_ref, bf1_ref, wf2_ref, bf2_ref, out_ref):
    a = aggp_ref[...]
    x2 = jnp.maximum(
        (a[0, :n] + a[1, :n] + xs2_ref[...]) * dinv_ref[...] + b2_ref[...], 0.0)
    gid = lax.broadcasted_iota(jnp.int32, (g, 1), 0)
    oh = (gid == batch_ref[...]).astype(jnp.float32)
    sums1 = jnp.dot(oh, x1_ref[...], preferred_element_type=jnp.float32,
                 precision=lax.Precision.HIGHEST)
    sums2 = jnp.dot(oh, x2, preferred_element_type=jnp.float32,
                 precision=lax.Precision.HIGHEST)
    inv_cnt = 1.0 / jnp.maximum(jnp.sum(oh, axis=1, keepdims=True), 1.0)
    pool1 = sums1 * inv_cnt
    pool2 = sums2 * inv_cnt
    h = jnp.maximum(
        jnp.dot(pool1, wf1a_ref[...], preferred_element_type=jnp.float32,
                 precision=lax.Precision.HIGHEST)
        + jnp.dot(pool2, wf1b_ref[...], preferred_element_type=jnp.float32,
                 precision=lax.Precision.HIGHEST)
        + bf1_ref[...], 0.0)
    out_ref[...] = jnp.dot(h, wf2_ref[...],
                           preferred_element_type=jnp.float32,
                 precision=lax.Precision.HIGHEST) + bf2_ref[...]


def kernel(x, edge_index, batch, W1, b1, W2, b2, Wf1, bf1, Wf2, bf2):
    n, d = x.shape
    e = edge_index.shape[1]
    f1 = W1.shape[1]
    f2 = W2.shape[1]
    g = 64
    np_pad = ((n + NS * 8 - 1) // (NS * 8)) * (NS * 8)
    n_chunks = e // B
    chunks_per_tile = n_chunks // NW

    src2d = edge_index[0].reshape(n_chunks, B)
    dst2d = edge_index[1].reshape(n_chunks, B)
    batch2d = batch.reshape(1, n)
    ones_b = jnp.ones((B, DEGW), jnp.float32)
    zeros1 = jnp.zeros((np_pad, DEGW), jnp.float32)
    zeros_f1 = jnp.zeros((np_pad, f1), jnp.float32)
    zeros_f2 = jnp.zeros((np_pad, f2), jnp.float32)

    deg_k = _make_deg_kernel(n_chunks, chunks_per_tile, np_pad)
    degp = deg_k(dst2d, ones_b, zeros1)

    scale1 = pl.pallas_call(
        functools.partial(_scale1_body, n),
        out_shape=(jax.ShapeDtypeStruct((n, f1), jnp.float32),
                   jax.ShapeDtypeStruct((n, 1), jnp.float32)),
    )
    xs1, dinv = scale1(degp, x, W1)

    conv1_k = _make_conv_kernel(f1, n_chunks, chunks_per_tile, np_pad)
    agg1p = conv1_k(xs1, src2d, dst2d, zeros_f1)

    mid = pl.pallas_call(
        functools.partial(_mid_body, n),
        out_shape=(jax.ShapeDtypeStruct((n, f1), jnp.float32),
                   jax.ShapeDtypeStruct((n, f2), jnp.float32)),
    )
    x1, xs2 = mid(agg1p, xs1, dinv, b1.reshape(1, f1), W2)

    conv2_k = _make_conv_kernel(f2, n_chunks, chunks_per_tile, np_pad)
    agg2p = conv2_k(xs2, src2d, dst2d, zeros_f2)

    final = pl.pallas_call(
        functools.partial(_final_body, n, g),
        out_shape=jax.ShapeDtypeStruct((g, 1), jnp.float32),
    )
    out = final(agg2p, xs2, dinv, b2.reshape(1, f2), x1, batch2d,
                Wf1[:f1], Wf1[f1:], bf1.reshape(1, -1), Wf2,
                bf2.reshape(1, 1))
    return out.reshape(-1)


# speedup vs baseline: 44.2429x; 1.0204x over previous
import functools

import jax
import jax.numpy as jnp
from jax import lax
from jax.experimental import pallas as pl
from jax.experimental.pallas import tpu as pltpu
from jax.experimental.pallas import tpu_sc as plsc

NC = 2
NS = 16
NW = NC * NS
B = 125


def _sc_mesh():
    return plsc.VectorSubcoreMesh(core_axis_name="c", subcore_axis_name="s",
                                  num_cores=NC, num_subcores=NS)


DEGW = 16


def _make_deg_kernel(n_chunks, chunks_per_tile, np_pad):
    rows = np_pad // NS

    @functools.partial(
        pl.kernel,
        mesh=_sc_mesh(),
        compiler_params=pltpu.CompilerParams(use_tc_tiling_on_sc=False),
        out_type=jax.ShapeDtypeStruct((NC, np_pad, DEGW), jnp.float32),
        scratch_types=[
            pltpu.VMEM((chunks_per_tile, B), jnp.int32),
            pltpu.VMEM((B, DEGW), jnp.float32),
            pltpu.VMEM_SHARED((np_pad, DEGW), jnp.float32),
            pltpu.SemaphoreType.DMA,
        ],
    )
    def deg_kernel(dst_hbm, ones_hbm, zeros_hbm, out_hbm, idx_v, ones_v, acc_sh,
                   sem):
        c = lax.axis_index("c")
        s = lax.axis_index("s")
        wid = s * NC + c
        pltpu.sync_copy(zeros_hbm.at[pl.ds(s * rows, rows)],
                        acc_sh.at[pl.ds(s * rows, rows)])
        pltpu.sync_copy(ones_hbm, ones_v)
        pltpu.sync_copy(dst_hbm.at[pl.ds(wid * chunks_per_tile, chunks_per_tile)],
                        idx_v)
        plsc.subcore_barrier()

        def fire(i, carry):
            pltpu.async_copy(ones_v, acc_sh.at[idx_v.at[i]], sem, add=True)
            return carry

        lax.fori_loop(0, chunks_per_tile, fire, 0)

        def drain(i, carry):
            pltpu.make_async_copy(ones_v, acc_sh.at[idx_v.at[i]], sem).wait()
            return carry

        lax.fori_loop(0, chunks_per_tile, drain, 0)
        plsc.subcore_barrier()
        pltpu.sync_copy(acc_sh.at[pl.ds(s * rows, rows)],
                        out_hbm.at[c, pl.ds(s * rows, rows)])

    return deg_kernel


def _make_conv_kernel(feat, n_chunks, chunks_per_tile, np_pad, nbuf):
    NBUF = nbuf
    rows = np_pad // NS
    n_batches = chunks_per_tile // NBUF

    @functools.partial(
        pl.kernel,
        mesh=_sc_mesh(),
        compiler_params=pltpu.CompilerParams(use_tc_tiling_on_sc=False),
        out_type=jax.ShapeDtypeStruct((NC, np_pad, feat), jnp.float32),
        scratch_types=[
            pltpu.VMEM((chunks_per_tile, B), jnp.int32),
            pltpu.VMEM((chunks_per_tile, B), jnp.int32),
            pltpu.VMEM((2, NBUF, B, feat), jnp.float32),
            pltpu.VMEM_SHARED((np_pad, feat), jnp.float32),
            pltpu.SemaphoreType.DMA,
            pltpu.SemaphoreType.DMA,
            pltpu.SemaphoreType.DMA,
        ],
    )
    def conv_kernel(table_hbm, src_hbm, dst_hbm, zeros_hbm, out_hbm,
                    idxs_v, idxd_v, bufs_v, acc_sh, gsem, ssem_a, ssem_b):
        ssems = (ssem_a, ssem_b)
        c = lax.axis_index("c")
        s = lax.axis_index("s")
        wid = s * NC + c
        pltpu.sync_copy(zeros_hbm.at[pl.ds(s * rows, rows)],
                        acc_sh.at[pl.ds(s * rows, rows)])
        pltpu.sync_copy(src_hbm.at[pl.ds(wid * chunks_per_tile, chunks_per_tile)],
                        idxs_v)
        pltpu.sync_copy(dst_hbm.at[pl.ds(wid * chunks_per_tile, chunks_per_tile)],
                        idxd_v)
        plsc.subcore_barrier()

        def g_start(t, grp, b):
            pltpu.async_copy(table_hbm.at[idxs_v.at[t * NBUF + b]],
                             bufs_v.at[grp, b], gsem)

        def g_wait(t, grp, b):
            pltpu.make_async_copy(table_hbm.at[idxs_v.at[t * NBUF + b]],
                                  bufs_v.at[grp, b], gsem).wait()

        def s_start(t, grp, b):
            pltpu.async_copy(bufs_v.at[grp, b],
                             acc_sh.at[idxd_v.at[t * NBUF + b]], ssems[grp],
                             add=True)

        def s_wait(t, grp, b):
            pltpu.make_async_copy(bufs_v.at[grp, b],
                                  acc_sh.at[idxd_v.at[t * NBUF + b]],
                                  ssems[grp]).wait()

        for b in range(NBUF):
            g_start(0, 0, b)

        def outer(t2, carry):
            for p in (0, 1):
                t = t2 * 2 + p
                for b in range(NBUF):
                    g_wait(t, p, b)
                for b in range(NBUF):
                    s_start(t, p, b)

                @pl.when(t >= 1)
                def _():
                    for b in range(NBUF):
                        s_wait(t - 1, 1 - p, b)

                @pl.when(t + 1 < n_batches)
                def _():
                    for b in range(NBUF):
                        g_start(t + 1, 1 - p, b)
            return carry

        lax.fori_loop(0, n_batches // 2, outer, 0)
        for b in range(NBUF):
            s_wait(n_batches - 1, 1, b)
        plsc.subcore_barrier()
        pltpu.sync_copy(acc_sh.at[pl.ds(s * rows, rows)],
                        out_hbm.at[c, pl.ds(s * rows, rows)])

    return conv_kernel


def _mm1_body(x_ref, w1_ref, xw_ref):
    xw_ref[...] = jnp.dot(x_ref[...], w1_ref[...],
                          preferred_element_type=jnp.float32,
                          precision=lax.Precision.HIGHEST)


def _scale1_body(n, degp_ref, xw_ref, xs_ref, dinv_ref):
    dp = degp_ref[...]
    deg = dp[0, :n, :1] + dp[1, :n, :1] + 1.0
    dinv = lax.rsqrt(deg)
    xs_ref[...] = xw_ref[...] * dinv
    dinv_ref[...] = dinv


def _mid_body(n, aggp_ref, xs_ref, dinv_ref, b1_ref, w2_ref, x1_ref, xs2_ref):
    a = aggp_ref[...]
    dinv = dinv_ref[...]
    x1 = jnp.maximum((a[0, :n] + a[1, :n] + xs_ref[...]) * dinv + b1_ref[...], 0.0)
    x1_ref[...] = x1
    xs2_ref[...] = jnp.dot(x1, w2_ref[...],
                           preferred_element_type=jnp.float32,
                 precision=lax.Precision.HIGHEST) * dinv


def _final_body(n, g, aggp_ref, xs2_ref, dinv_ref, b2_ref, x1_ref, batch_ref,
                wf1a_ref, wf1b_ref, bf1_ref, wf2_ref, bf2_ref, out_ref):
    a = aggp_ref[...]
    x2 = jnp.maximum(
        (a[0, :n] + a[1, :n] + xs2_ref[...]) * dinv_ref[...] + b2_ref[...], 0.0)
    gid = lax.broadcasted_iota(jnp.int32, (g, 1), 0)
    oh = (gid == batch_ref[...]).astype(jnp.float32)
    sums1 = jnp.dot(oh, x1_ref[...], preferred_element_type=jnp.float32,
                 precision=lax.Precision.HIGHEST)
    sums2 = jnp.dot(oh, x2, preferred_element_type=jnp.float32,
                 precision=lax.Precision.HIGHEST)
    inv_cnt = 1.0 / jnp.maximum(jnp.sum(oh, axis=1, keepdims=True), 1.0)
    pool1 = sums1 * inv_cnt
    pool2 = sums2 * inv_cnt
    h = jnp.maximum(
        jnp.dot(pool1, wf1a_ref[...], preferred_element_type=jnp.float32,
                 precision=lax.Precision.HIGHEST)
        + jnp.dot(pool2, wf1b_ref[...], preferred_element_type=jnp.float32,
                 precision=lax.Precision.HIGHEST)
        + bf1_ref[...], 0.0)
    out_ref[...] = jnp.dot(h, wf2_ref[...],
                           preferred_element_type=jnp.float32,
                 precision=lax.Precision.HIGHEST) + bf2_ref[...]


def kernel(x, edge_index, batch, W1, b1, W2, b2, Wf1, bf1, Wf2, bf2):
    n, d = x.shape
    e = edge_index.shape[1]
    f1 = W1.shape[1]
    f2 = W2.shape[1]
    g = 64
    np_pad = ((n + NS * 8 - 1) // (NS * 8)) * (NS * 8)
    n_chunks = e // B
    chunks_per_tile = n_chunks // NW

    src2d = edge_index[0].reshape(n_chunks, B)
    dst2d = edge_index[1].reshape(n_chunks, B)
    batch2d = batch.reshape(1, n)
    ones_b = jnp.ones((B, DEGW), jnp.float32)
    zeros1 = jnp.zeros((np_pad, DEGW), jnp.float32)
    zeros_f1 = jnp.zeros((np_pad, f1), jnp.float32)
    zeros_f2 = jnp.zeros((np_pad, f2), jnp.float32)

    deg_k = _make_deg_kernel(n_chunks, chunks_per_tile, np_pad)
    degp = deg_k(dst2d, ones_b, zeros1)

    mm1 = pl.pallas_call(
        _mm1_body, out_shape=jax.ShapeDtypeStruct((n, f1), jnp.float32))
    xw1 = mm1(x, W1)

    scale1 = pl.pallas_call(
        functools.partial(_scale1_body, n),
        out_shape=(jax.ShapeDtypeStruct((n, f1), jnp.float32),
                   jax.ShapeDtypeStruct((n, 1), jnp.float32)),
    )
    xs1, dinv = scale1(degp, xw1)

    conv1_k = _make_conv_kernel(f1, n_chunks, chunks_per_tile, np_pad, 8)
    agg1p = conv1_k(xs1, src2d, dst2d, zeros_f1)

    mid = pl.pallas_call(
        functools.partial(_mid_body, n),
        out_shape=(jax.ShapeDtypeStruct((n, f1), jnp.float32),
                   jax.ShapeDtypeStruct((n, f2), jnp.float32)),
    )
    x1, xs2 = mid(agg1p, xs1, dinv, b1.reshape(1, f1), W2)

    conv2_k = _make_conv_kernel(f2, n_chunks, chunks_per_tile, np_pad, 4)
    agg2p = conv2_k(xs2, src2d, dst2d, zeros_f2)

    final = pl.pallas_call(
        functools.partial(_final_body, n, g),
        out_shape=jax.ShapeDtypeStruct((g, 1), jnp.float32),
    )
    out = final(agg2p, xs2, dinv, b2.reshape(1, f2), x1, batch2d,
                Wf1[:f1], Wf1[f1:], bf1.reshape(1, -1), Wf2,
                bf2.reshape(1, 1))
    return out.reshape(-1)


# speedup vs baseline: 44.9241x; 1.0154x over previous
import functools

import jax
import jax.numpy as jnp
from jax import lax
from jax.experimental import pallas as pl
from jax.experimental.pallas import tpu as pltpu
from jax.experimental.pallas import tpu_sc as plsc

NC = 2
NS = 16
NW = NC * NS
B = 125


def _sc_mesh():
    return plsc.VectorSubcoreMesh(core_axis_name="c", subcore_axis_name="s",
                                  num_cores=NC, num_subcores=NS)


DEGW = 8


def _make_deg_kernel(n_chunks, chunks_per_tile, np_pad):
    rows = np_pad // NS

    @functools.partial(
        pl.kernel,
        mesh=_sc_mesh(),
        compiler_params=pltpu.CompilerParams(use_tc_tiling_on_sc=False),
        out_type=jax.ShapeDtypeStruct((NC, np_pad, DEGW), jnp.float32),
        scratch_types=[
            pltpu.VMEM((chunks_per_tile, B), jnp.int32),
            pltpu.VMEM((B, DEGW), jnp.float32),
            pltpu.VMEM_SHARED((np_pad, DEGW), jnp.float32),
            pltpu.SemaphoreType.DMA,
        ],
    )
    def deg_kernel(dst_hbm, ones_hbm, zeros_hbm, out_hbm, idx_v, ones_v, acc_sh,
                   sem):
        c = lax.axis_index("c")
        s = lax.axis_index("s")
        wid = s * NC + c
        pltpu.sync_copy(zeros_hbm.at[pl.ds(s * rows, rows)],
                        acc_sh.at[pl.ds(s * rows, rows)])
        pltpu.sync_copy(ones_hbm, ones_v)
        pltpu.sync_copy(dst_hbm.at[pl.ds(wid * chunks_per_tile, chunks_per_tile)],
                        idx_v)
        plsc.subcore_barrier()

        def fire(i, carry):
            pltpu.async_copy(ones_v, acc_sh.at[idx_v.at[i]], sem, add=True)
            return carry

        lax.fori_loop(0, chunks_per_tile, fire, 0)

        def drain(i, carry):
            pltpu.make_async_copy(ones_v, acc_sh.at[idx_v.at[i]], sem).wait()
            return carry

        lax.fori_loop(0, chunks_per_tile, drain, 0)
        plsc.subcore_barrier()
        pltpu.sync_copy(acc_sh.at[pl.ds(s * rows, rows)],
                        out_hbm.at[c, pl.ds(s * rows, rows)])

    return deg_kernel


def _make_conv_kernel(feat, n_chunks, chunks_per_tile, np_pad, nbuf):
    NBUF = nbuf
    rows = np_pad // NS
    n_batches = chunks_per_tile // NBUF

    @functools.partial(
        pl.kernel,
        mesh=_sc_mesh(),
        compiler_params=pltpu.CompilerParams(use_tc_tiling_on_sc=False),
        out_type=jax.ShapeDtypeStruct((NC, np_pad, feat), jnp.float32),
        scratch_types=[
            pltpu.VMEM((chunks_per_tile, B), jnp.int32),
            pltpu.VMEM((chunks_per_tile, B), jnp.int32),
            pltpu.VMEM((2, NBUF, B, feat), jnp.float32),
            pltpu.VMEM_SHARED((np_pad, feat), jnp.float32),
            pltpu.SemaphoreType.DMA,
            pltpu.SemaphoreType.DMA,
            pltpu.SemaphoreType.DMA,
        ],
    )
    def conv_kernel(table_hbm, src_hbm, dst_hbm, zeros_hbm, out_hbm,
                    idxs_v, idxd_v, bufs_v, acc_sh, gsem, ssem_a, ssem_b):
        ssems = (ssem_a, ssem_b)
        c = lax.axis_index("c")
        s = lax.axis_index("s")
        wid = s * NC + c
        z = pltpu.async_copy(zeros_hbm.at[pl.ds(s * rows, rows)],
                             acc_sh.at[pl.ds(s * rows, rows)], gsem)
        i1 = pltpu.async_copy(
            src_hbm.at[pl.ds(wid * chunks_per_tile, chunks_per_tile)], idxs_v,
            ssem_a)
        i2 = pltpu.async_copy(
            dst_hbm.at[pl.ds(wid * chunks_per_tile, chunks_per_tile)], idxd_v,
            ssem_b)
        z.wait()
        i1.wait()
        i2.wait()
        plsc.subcore_barrier()

        def g_start(t, grp, b):
            pltpu.async_copy(table_hbm.at[idxs_v.at[t * NBUF + b]],
                             bufs_v.at[grp, b], gsem)

        def g_wait(t, grp, b):
            pltpu.make_async_copy(table_hbm.at[idxs_v.at[t * NBUF + b]],
                                  bufs_v.at[grp, b], gsem).wait()

        def s_start(t, grp, b):
            pltpu.async_copy(bufs_v.at[grp, b],
                             acc_sh.at[idxd_v.at[t * NBUF + b]], ssems[grp],
                             add=True)

        def s_wait(t, grp, b):
            pltpu.make_async_copy(bufs_v.at[grp, b],
                                  acc_sh.at[idxd_v.at[t * NBUF + b]],
                                  ssems[grp]).wait()

        for b in range(NBUF):
            g_start(0, 0, b)

        def outer(t2, carry):
            for p in (0, 1):
                t = t2 * 2 + p
                for b in range(NBUF):
                    g_wait(t, p, b)
                for b in range(NBUF):
                    s_start(t, p, b)

                @pl.when(t >= 1)
                def _():
                    for b in range(NBUF):
                        s_wait(t - 1, 1 - p, b)

                @pl.when(t + 1 < n_batches)
                def _():
                    for b in range(NBUF):
                        g_start(t + 1, 1 - p, b)
            return carry

        lax.fori_loop(0, n_batches // 2, outer, 0)
        for b in range(NBUF):
            s_wait(n_batches - 1, 1, b)
        plsc.subcore_barrier()
        pltpu.sync_copy(acc_sh.at[pl.ds(s * rows, rows)],
                        out_hbm.at[c, pl.ds(s * rows, rows)])

    return conv_kernel


def _mm1_body(x_ref, w1_ref, xw_ref):
    xw_ref[...] = jnp.dot(x_ref[...], w1_ref[...],
                          preferred_element_type=jnp.float32,
                          precision=lax.Precision.HIGHEST)


def _scale1_body(n, degp_ref, xw_ref, xs_ref, dinv_ref):
    dp = degp_ref[...]
    deg = dp[0, :n, :1] + dp[1, :n, :1] + 1.0
    dinv = lax.rsqrt(deg)
    xs_ref[...] = xw_ref[...] * dinv
    dinv_ref[...] = dinv


def _mid_body(n, aggp_ref, xs_ref, dinv_ref, b1_ref, w2_ref, x1_ref, xs2_ref):
    a = aggp_ref[...]
    dinv = dinv_ref[...]
    x1 = jnp.maximum((a[0, :n] + a[1, :n] + xs_ref[...]) * dinv + b1_ref[...], 0.0)
    x1_ref[...] = x1
    xs2_ref[...] = jnp.dot(x1, w2_ref[...],
                           preferred_element_type=jnp.float32,
                 precision=lax.Precision.HIGHEST) * dinv


def _final_body(n, g, aggp_ref, xs2_ref, dinv_ref, b2_ref, x1_ref, batch_ref,
                wf1a_ref, wf1b_ref, bf1_ref, wf2_ref, bf2_ref, out_ref):
    a = aggp_ref[...]
    x2 = jnp.maximum(
        (a[0, :n] + a[1, :n] + xs2_ref[...]) * dinv_ref[...] + b2_ref[...], 0.0)
    gid = lax.broadcasted_iota(jnp.int32, (g, 1), 0)
    oh = (gid == batch_ref[...]).astype(jnp.float32)
    sums1 = jnp.dot(oh, x1_ref[...], preferred_element_type=jnp.float32,
                 precision=lax.Precision.HIGHEST)
    sums2 = jnp.dot(oh, x2, preferred_element_type=jnp.float32,
                 precision=lax.Precision.HIGHEST)
    inv_cnt = 1.0 / jnp.maximum(jnp.sum(oh, axis=1, keepdims=True), 1.0)
    pool1 = sums1 * inv_cnt
    pool2 = sums2 * inv_cnt
    h = jnp.maximum(
        jnp.dot(pool1, wf1a_ref[...], preferred_element_type=jnp.float32,
                 precision=lax.Precision.HIGHEST)
        + jnp.dot(pool2, wf1b_ref[...], preferred_element_type=jnp.float32,
                 precision=lax.Precision.HIGHEST)
        + bf1_ref[...], 0.0)
    out_ref[...] = jnp.dot(h, wf2_ref[...],
                           preferred_element_type=jnp.float32,
                 precision=lax.Precision.HIGHEST) + bf2_ref[...]


def kernel(x, edge_index, batch, W1, b1, W2, b2, Wf1, bf1, Wf2, bf2):
    n, d = x.shape
    e = edge_index.shape[1]
    f1 = W1.shape[1]
    f2 = W2.shape[1]
    g = 64
    np_pad = ((n + NS * 8 - 1) // (NS * 8)) * (NS * 8)
    n_chunks = e // B
    chunks_per_tile = n_chunks // NW

    src2d = edge_index[0].reshape(n_chunks, B)
    dst2d = edge_index[1].reshape(n_chunks, B)
    batch2d = batch.reshape(1, n)
    ones_b = jnp.ones((B, DEGW), jnp.float32)
    zeros1 = jnp.zeros((np_pad, DEGW), jnp.float32)
    zeros_f1 = jnp.zeros((np_pad, f1), jnp.float32)
    zeros_f2 = jnp.zeros((np_pad, f2), jnp.float32)

    deg_k = _make_deg_kernel(n_chunks, chunks_per_tile, np_pad)
    degp = deg_k(dst2d, ones_b, zeros1)

    mm1 = pl.pallas_call(
        _mm1_body, out_shape=jax.ShapeDtypeStruct((n, f1), jnp.float32))
    xw1 = mm1(x, W1)

    scale1 = pl.pallas_call(
        functools.partial(_scale1_body, n),
        out_shape=(jax.ShapeDtypeStruct((n, f1), jnp.float32),
                   jax.ShapeDtypeStruct((n, 1), jnp.float32)),
    )
    xs1, dinv = scale1(degp, xw1)

    conv1_k = _make_conv_kernel(f1, n_chunks, chunks_per_tile, np_pad, 8)
    agg1p = conv1_k(xs1, src2d, dst2d, zeros_f1)

    mid = pl.pallas_call(
        functools.partial(_mid_body, n),
        out_shape=(jax.ShapeDtypeStruct((n, f1), jnp.float32),
                   jax.ShapeDtypeStruct((n, f2), jnp.float32)),
    )
    x1, xs2 = mid(agg1p, xs1, dinv, b1.reshape(1, f1), W2)

    conv2_k = _make_conv_kernel(f2, n_chunks, chunks_per_tile, np_pad, 4)
    agg2p = conv2_k(xs2, src2d, dst2d, zeros_f2)

    final = pl.pallas_call(
        functools.partial(_final_body, n, g),
        out_shape=jax.ShapeDtypeStruct((g, 1), jnp.float32),
    )
    out = final(agg2p, xs2, dinv, b2.reshape(1, f2), x1, batch2d,
                Wf1[:f1], Wf1[f1:], bf1.reshape(1, -1), Wf2,
                bf2.reshape(1, 1))
    return out.reshape(-1)


# speedup vs baseline: 45.0228x; 1.0022x over previous
import functools

import jax
import jax.numpy as jnp
from jax import lax
from jax.experimental import pallas as pl
from jax.experimental.pallas import tpu as pltpu
from jax.experimental.pallas import tpu_sc as plsc

NC = 2
NS = 16
NW = NC * NS
B = 125


def _sc_mesh():
    return plsc.VectorSubcoreMesh(core_axis_name="c", subcore_axis_name="s",
                                  num_cores=NC, num_subcores=NS)


DEGW = 16


def _make_deg_kernel(n_chunks, chunks_per_tile, np_pad):
    rows = np_pad // NS

    @functools.partial(
        pl.kernel,
        mesh=_sc_mesh(),
        compiler_params=pltpu.CompilerParams(use_tc_tiling_on_sc=False),
        out_type=jax.ShapeDtypeStruct((NC, np_pad, DEGW), jnp.float32),
        scratch_types=[
            pltpu.VMEM((chunks_per_tile, B), jnp.int32),
            pltpu.VMEM((B, DEGW), jnp.float32),
            pltpu.VMEM_SHARED((np_pad, DEGW), jnp.float32),
            pltpu.SemaphoreType.DMA,
        ],
    )
    def deg_kernel(dst_hbm, ones_hbm, zeros_hbm, out_hbm, idx_v, ones_v, acc_sh,
                   sem):
        c = lax.axis_index("c")
        s = lax.axis_index("s")
        wid = s * NC + c
        pltpu.sync_copy(zeros_hbm.at[pl.ds(s * rows, rows)],
                        acc_sh.at[pl.ds(s * rows, rows)])
        pltpu.sync_copy(ones_hbm, ones_v)
        pltpu.sync_copy(dst_hbm.at[pl.ds(wid * chunks_per_tile, chunks_per_tile)],
                        idx_v)
        plsc.subcore_barrier()

        def fire(i, carry):
            pltpu.async_copy(ones_v, acc_sh.at[idx_v.at[i]], sem, add=True)
            return carry

        lax.fori_loop(0, chunks_per_tile, fire, 0)

        def drain(i, carry):
            pltpu.make_async_copy(ones_v, acc_sh.at[idx_v.at[i]], sem).wait()
            return carry

        lax.fori_loop(0, chunks_per_tile, drain, 0)
        plsc.subcore_barrier()
        pltpu.sync_copy(acc_sh.at[pl.ds(s * rows, rows)],
                        out_hbm.at[c, pl.ds(s * rows, rows)])

    return deg_kernel


def _make_conv_kernel(feat, n_chunks, chunks_per_tile, np_pad, nbuf):
    NBUF = nbuf
    rows = np_pad // NS
    n_batches = chunks_per_tile // NBUF

    @functools.partial(
        pl.kernel,
        mesh=_sc_mesh(),
        compiler_params=pltpu.CompilerParams(use_tc_tiling_on_sc=False),
        out_type=jax.ShapeDtypeStruct((NC, np_pad, feat), jnp.float32),
        scratch_types=[
            pltpu.VMEM((chunks_per_tile, B), jnp.int32),
            pltpu.VMEM((chunks_per_tile, B), jnp.int32),
            pltpu.VMEM((2, NBUF, B, feat), jnp.float32),
            pltpu.VMEM_SHARED((np_pad, feat), jnp.float32),
            pltpu.SemaphoreType.DMA,
            pltpu.SemaphoreType.DMA,
            pltpu.SemaphoreType.DMA,
        ],
    )
    def conv_kernel(table_hbm, src_hbm, dst_hbm, zeros_hbm, out_hbm,
                    idxs_v, idxd_v, bufs_v, acc_sh, gsem, ssem_a, ssem_b):
        ssems = (ssem_a, ssem_b)
        c = lax.axis_index("c")
        s = lax.axis_index("s")
        wid = s * NC + c
        z = pltpu.async_copy(zeros_hbm.at[pl.ds(s * rows, rows)],
                             acc_sh.at[pl.ds(s * rows, rows)], gsem)
        i1 = pltpu.async_copy(
            src_hbm.at[pl.ds(wid * chunks_per_tile, chunks_per_tile)], idxs_v,
            ssem_a)
        i2 = pltpu.async_copy(
            dst_hbm.at[pl.ds(wid * chunks_per_tile, chunks_per_tile)], idxd_v,
            ssem_b)
        z.wait()
        i1.wait()
        i2.wait()
        plsc.subcore_barrier()

        def g_start(t, grp, b):
            pltpu.async_copy(table_hbm.at[idxs_v.at[t * NBUF + b]],
                             bufs_v.at[grp, b], gsem)

        def g_wait(t, grp, b):
            pltpu.make_async_copy(table_hbm.at[idxs_v.at[t * NBUF + b]],
                                  bufs_v.at[grp, b], gsem).wait()

        def s_start(t, grp, b):
            pltpu.async_copy(bufs_v.at[grp, b],
                             acc_sh.at[idxd_v.at[t * NBUF + b]], ssems[grp],
                             add=True)

        def s_wait(t, grp, b):
            pltpu.make_async_copy(bufs_v.at[grp, b],
                                  acc_sh.at[idxd_v.at[t * NBUF + b]],
                                  ssems[grp]).wait()

        for b in range(NBUF):
            g_start(0, 0, b)

        def outer(t2, carry):
            for p in (0, 1):
                t = t2 * 2 + p
                for b in range(NBUF):
                    g_wait(t, p, b)
                for b in range(NBUF):
                    s_start(t, p, b)

                @pl.when(t >= 1)
                def _():
                    for b in range(NBUF):
                        s_wait(t - 1, 1 - p, b)

                @pl.when(t + 1 < n_batches)
                def _():
                    for b in range(NBUF):
                        g_start(t + 1, 1 - p, b)
            return carry

        lax.fori_loop(0, n_batches // 2, outer, 0)
        for b in range(NBUF):
            s_wait(n_batches - 1, 1, b)
        plsc.subcore_barrier()
        pltpu.sync_copy(acc_sh.at[pl.ds(s * rows, rows)],
                        out_hbm.at[c, pl.ds(s * rows, rows)])

    return conv_kernel


def _mm1_body(x_ref, w1_ref, xw_ref):
    xw_ref[...] = jnp.dot(x_ref[...], w1_ref[...],
                          preferred_element_type=jnp.float32,
                          precision=lax.Precision.HIGHEST)


def _scale1_body(n, degp_ref, xw_ref, xs_ref, dinv_ref):
    dp = degp_ref[...]
    deg = dp[0, :n, :1] + dp[1, :n, :1] + 1.0
    dinv = lax.rsqrt(deg)
    xs_ref[...] = xw_ref[...] * dinv
    dinv_ref[...] = dinv


def _mid_body(n, aggp_ref, xs_ref, dinv_ref, b1_ref, w2_ref, x1_ref, xs2_ref):
    a = aggp_ref[...]
    dinv = dinv_ref[...]
    x1 = jnp.maximum((a[0, :n] + a[1, :n] + xs_ref[...]) * dinv + b1_ref[...], 0.0)
    x1_ref[...] = x1
    xs2_ref[...] = jnp.dot(x1, w2_ref[...],
                           preferred_element_type=jnp.float32,
                 precision=lax.Precision.HIGHEST) * dinv


def _final_body(n, g, aggp_ref, xs2_ref, dinv_ref, b2_ref, x1_ref, batch_ref,
                wf1a_ref, wf1b_ref, bf1_ref, wf2_ref, bf2_ref, out_ref):
    a = aggp_ref[...]
    x2 = jnp.maximum(
        (a[0, :n] + a[1, :n] + xs2_ref[...]) * dinv_ref[...] + b2_ref[...], 0.0)
    gid = lax.broadcasted_iota(jnp.int32, (g, 1), 0)
    oh = (gid == batch_ref[...]).astype(jnp.float32)
    sums1 = jnp.dot(oh, x1_ref[...], preferred_element_type=jnp.float32,
                 precision=lax.Precision.HIGHEST)
    sums2 = jnp.dot(oh, x2, preferred_element_type=jnp.float32,
                 precision=lax.Precision.HIGHEST)
    inv_cnt = 1.0 / jnp.maximum(jnp.sum(oh, axis=1, keepdims=True), 1.0)
    pool1 = sums1 * inv_cnt
    pool2 = sums2 * inv_cnt
    h = jnp.maximum(
        jnp.dot(pool1, wf1a_ref[...], preferred_element_type=jnp.float32,
                 precision=lax.Precision.HIGHEST)
        + jnp.dot(pool2, wf1b_ref[...], preferred_element_type=jnp.float32,
                 precision=lax.Precision.HIGHEST)
        + bf1_ref[...], 0.0)
    out_ref[...] = jnp.dot(h, wf2_ref[...],
                           preferred_element_type=jnp.float32,
                 precision=lax.Precision.HIGHEST) + bf2_ref[...]


def kernel(x, edge_index, batch, W1, b1, W2, b2, Wf1, bf1, Wf2, bf2):
    n, d = x.shape
    e = edge_index.shape[1]
    f1 = W1.shape[1]
    f2 = W2.shape[1]
    g = 64
    np_pad = ((n + NS * 8 - 1) // (NS * 8)) * (NS * 8)
    n_chunks = e // B
    chunks_per_tile = n_chunks // NW

    src2d = edge_index[0].reshape(n_chunks, B)
    dst2d = edge_index[1].reshape(n_chunks, B)
    batch2d = batch.reshape(1, n)
    ones_b = jnp.ones((B, DEGW), jnp.float32)
    zeros1 = jnp.zeros((np_pad, DEGW), jnp.float32)
    zeros_f1 = jnp.zeros((np_pad, f1), jnp.float32)
    zeros_f2 = jnp.zeros((np_pad, f2), jnp.float32)

    deg_k = _make_deg_kernel(n_chunks, chunks_per_tile, np_pad)
    degp = deg_k(dst2d, ones_b, zeros1)

    mm1 = pl.pallas_call(
        _mm1_body, out_shape=jax.ShapeDtypeStruct((n, f1), jnp.float32))
    xw1 = mm1(x, W1)

    scale1 = pl.pallas_call(
        functools.partial(_scale1_body, n),
        out_shape=(jax.ShapeDtypeStruct((n, f1), jnp.float32),
                   jax.ShapeDtypeStruct((n, 1), jnp.float32)),
    )
    xs1, dinv = scale1(degp, xw1)

    conv1_k = _make_conv_kernel(f1, n_chunks, chunks_per_tile, np_pad, 8)
    agg1p = conv1_k(xs1, src2d, dst2d, zeros_f1)

    mid = pl.pallas_call(
        functools.partial(_mid_body, n),
        out_shape=(jax.ShapeDtypeStruct((n, f1), jnp.float32),
                   jax.ShapeDtypeStruct((n, f2), jnp.float32)),
    )
    x1, xs2 = mid(agg1p, xs1, dinv, b1.reshape(1, f1), W2)

    conv2_k = _make_conv_kernel(f2, n_chunks, chunks_per_tile, np_pad, 4)
    agg2p = conv2_k(xs2, src2d, dst2d, zeros_f2)

    final = pl.pallas_call(
        functools.partial(_final_body, n, g),
        out_shape=jax.ShapeDtypeStruct((g, 1), jnp.float32),
    )
    out = final(agg2p, xs2, dinv, b2.reshape(1, f2), x1, batch2d,
                Wf1[:f1], Wf1[f1:], bf1.reshape(1, -1), Wf2,
                bf2.reshape(1, 1))
    return out.reshape(-1)
